# baseline (speedup 1.0000x reference)
import functools

import jax
import jax.numpy as jnp
from jax import lax
from jax.experimental import pallas as pl
from jax.experimental.pallas import tpu as pltpu
from jax.experimental.pallas import tpu_sc as plsc

_B, _S, _D = 4096, 20, 768
_VOCAB, _NSEG = 4, 2
_NROWS = _VOCAB * _NSEG * _S
_BS = _B * _S
_NC, _NS, _L = 2, 16, 16
_NW = _NC * _NS
_BPW = _B // _NW
_PER_W = _BPW * _S
_CB = 64
_HALVES = _BPW // _CB
_NCH = _S * _HALVES
_NBUF = 2


def _table_body(tok_ref, sege_ref, pos_ref, gamma_ref, beta_ref, out_ref):
    rows = lax.broadcasted_iota(jnp.int32, (_NROWS, 1), 0)
    v = rows // (_NSEG * _S)
    g = (rows // _S) % _NSEG
    s = rows % _S
    ohv = (v == lax.broadcasted_iota(jnp.int32, (_NROWS, _VOCAB), 1)).astype(jnp.float32)
    ohg = (g == lax.broadcasted_iota(jnp.int32, (_NROWS, _NSEG), 1)).astype(jnp.float32)
    ohs = (s == lax.broadcasted_iota(jnp.int32, (_NROWS, _S), 1)).astype(jnp.float32)
    emb = (
        jnp.dot(ohv, tok_ref[...], preferred_element_type=jnp.float32)
        + jnp.dot(ohg, sege_ref[...], preferred_element_type=jnp.float32)
        + jnp.dot(ohs, pos_ref[...], preferred_element_type=jnp.float32)
    )
    mean = jnp.mean(emb, axis=-1, keepdims=True)
    cent = emb - mean
    var = jnp.mean(cent * cent, axis=-1, keepdims=True)
    normed = cent * lax.rsqrt(var + 1e-5)
    out_ref[...] = normed * gamma_ref[...] + beta_ref[...]


def _build_table(tok_embed, pos20, seg_embed, gamma, beta):
    return pl.pallas_call(
        _table_body,
        out_shape=jax.ShapeDtypeStruct((_NROWS, _D), jnp.float32),
    )(tok_embed, seg_embed, pos20, gamma.reshape(1, _D), beta.reshape(1, _D))


def _sc_body(table_hbm, xt_hbm, st_hbm, out_hbm, xb, sb, idx, bufs, gsems, ssems):
    wid = lax.axis_index("s") * _NC + lax.axis_index("c")
    b0 = wid * _BPW
    hx = pltpu.async_copy(xt_hbm.at[:, pl.ds(b0, _BPW)], xb, gsems[0])
    hs = pltpu.async_copy(st_hbm.at[:, pl.ds(b0, _BPW)], sb, gsems[1])
    hx.wait()
    hs.wait()

    for c in range(_NCH):
        s, h = c % _S, c // _S
        for k in range(_CB // _L):
            o = h * _CB + k * _L
            cid = xb[s, pl.ds(o, _L)] * (_NSEG * _S) + sb[s, pl.ds(o, _L)] * _S + s
            idx[c, pl.ds(k * _L, _L)] = cid

    gather_h = [None] * _NCH
    store_h = [None] * _NCH
    for g in range(_NCH + _NBUF - 1):
        if g < _NCH:
            if g >= _NBUF:
                store_h[g - _NBUF].wait()
            gather_h[g] = pltpu.async_copy(
                table_hbm.at[idx.at[g]], bufs[g % _NBUF], gsems[g % _NBUF])
        d = g - (_NBUF - 1)
        if 0 <= d < _NCH:
            gather_h[d].wait()
            s, h = d % _S, d // _S
            store_h[d] = pltpu.async_copy(
                bufs[d % _NBUF],
                out_hbm.at[s, pl.ds(b0 + h * _CB, _CB)],
                ssems[d % _NBUF])
    for d in range(max(0, _NCH - _NBUF), _NCH):
        store_h[d].wait()


def _gather_rows(table, xt, st):
    mesh = plsc.VectorSubcoreMesh(
        core_axis_name="c", subcore_axis_name="s",
        num_cores=_NC, num_subcores=_NS)
    fn = functools.partial(
        pl.kernel,
        out_type=jax.ShapeDtypeStruct((_S, _B, _D), jnp.float32),
        mesh=mesh,
        scratch_types=[
            pltpu.VMEM((_S, _BPW), jnp.int32),
            pltpu.VMEM((_S, _BPW), jnp.int32),
            pltpu.VMEM((_NCH, _CB), jnp.int32),
            [pltpu.VMEM((_CB, _D), jnp.float32) for _ in range(_NBUF)],
            [pltpu.SemaphoreType.DMA for _ in range(_NBUF)],
            [pltpu.SemaphoreType.DMA for _ in range(_NBUF)],
        ],
    )(_sc_body)
    return fn(table, xt, st)


def kernel(x, seg, tok_embed, pos_embed, seg_embed, gamma, beta):
    table = _build_table(tok_embed, pos_embed[:_S], seg_embed, gamma, beta)
    xt = x.T.astype(jnp.int32)
    st = seg.T.astype(jnp.int32)
    out_t = _gather_rows(table, xt, st)
    return jnp.transpose(out_t, (1, 0, 2))


# speedup vs baseline: 1.4479x; 1.4479x over previous
import functools

import jax
import jax.numpy as jnp
from jax import lax
from jax.experimental import pallas as pl
from jax.experimental.pallas import tpu as pltpu
from jax.experimental.pallas import tpu_sc as plsc

_B, _S, _D = 4096, 20, 768
_VOCAB, _NSEG = 4, 2
_NROWS = _VOCAB * _NSEG * _S
_BS = _B * _S
_NC, _NS, _L = 2, 16, 16
_NW = _NC * _NS
_BPW = _B // _NW
_PER_W = _BPW * _S
_CB = 16
_HALVES = _BPW // _CB
_NCH = _S * _HALVES
_NBUF = 8


def _table_body(tok_ref, sege_ref, pos_ref, gamma_ref, beta_ref, out_ref):
    rows = lax.broadcasted_iota(jnp.int32, (_NROWS, 1), 0)
    v = rows // (_NSEG * _S)
    g = (rows // _S) % _NSEG
    s = rows % _S
    ohv = (v == lax.broadcasted_iota(jnp.int32, (_NROWS, _VOCAB), 1)).astype(jnp.float32)
    ohg = (g == lax.broadcasted_iota(jnp.int32, (_NROWS, _NSEG), 1)).astype(jnp.float32)
    ohs = (s == lax.broadcasted_iota(jnp.int32, (_NROWS, _S), 1)).astype(jnp.float32)
    emb = (
        jnp.dot(ohv, tok_ref[...], preferred_element_type=jnp.float32)
        + jnp.dot(ohg, sege_ref[...], preferred_element_type=jnp.float32)
        + jnp.dot(ohs, pos_ref[...], preferred_element_type=jnp.float32)
    )
    mean = jnp.mean(emb, axis=-1, keepdims=True)
    cent = emb - mean
    var = jnp.mean(cent * cent, axis=-1, keepdims=True)
    normed = cent * lax.rsqrt(var + 1e-5)
    out_ref[...] = normed * gamma_ref[...] + beta_ref[...]


def _build_table(tok_embed, pos20, seg_embed, gamma, beta):
    return pl.pallas_call(
        _table_body,
        out_shape=jax.ShapeDtypeStruct((_NROWS, _D), jnp.float32),
    )(tok_embed, seg_embed, pos20, gamma.reshape(1, _D), beta.reshape(1, _D))


def _sc_body(table_hbm, xt_hbm, st_hbm, out_hbm, xb, sb, idx, bufs, gsems, ssems):
    wid = lax.axis_index("s") * _NC + lax.axis_index("c")
    b0 = wid * _BPW
    hx = pltpu.async_copy(xt_hbm.at[:, pl.ds(b0, _BPW)], xb, gsems[0])
    hs = pltpu.async_copy(st_hbm.at[:, pl.ds(b0, _BPW)], sb, gsems[1])
    hx.wait()
    hs.wait()

    for c in range(_NCH):
        s, h = c % _S, c // _S
        for k in range(_CB // _L):
            o = h * _CB + k * _L
            cid = xb[s, pl.ds(o, _L)] * (_NSEG * _S) + sb[s, pl.ds(o, _L)] * _S + s
            idx[c, pl.ds(k * _L, _L)] = cid

    gather_h = [None] * _NCH
    store_h = [None] * _NCH
    for g in range(_NCH + _NBUF - 1):
        if g < _NCH:
            if g >= _NBUF:
                store_h[g - _NBUF].wait()
            gather_h[g] = pltpu.async_copy(
                table_hbm.at[idx.at[g]], bufs[g % _NBUF], gsems[g % _NBUF])
        d = g - (_NBUF - 1)
        if 0 <= d < _NCH:
            gather_h[d].wait()
            s, h = d % _S, d // _S
            store_h[d] = pltpu.async_copy(
                bufs[d % _NBUF],
                out_hbm.at[s, pl.ds(b0 + h * _CB, _CB)],
                ssems[d % _NBUF])
    for d in range(max(0, _NCH - _NBUF), _NCH):
        store_h[d].wait()


def _gather_rows(table, xt, st):
    mesh = plsc.VectorSubcoreMesh(
        core_axis_name="c", subcore_axis_name="s",
        num_cores=_NC, num_subcores=_NS)
    fn = functools.partial(
        pl.kernel,
        out_type=jax.ShapeDtypeStruct((_S, _B, _D), jnp.float32),
        mesh=mesh,
        scratch_types=[
            pltpu.VMEM((_S, _BPW), jnp.int32),
            pltpu.VMEM((_S, _BPW), jnp.int32),
            pltpu.VMEM((_NCH, _CB), jnp.int32),
            [pltpu.VMEM((_CB, _D), jnp.float32) for _ in range(_NBUF)],
            [pltpu.SemaphoreType.DMA for _ in range(_NBUF)],
            [pltpu.SemaphoreType.DMA for _ in range(_NBUF)],
        ],
    )(_sc_body)
    return fn(table, xt, st)


def kernel(x, seg, tok_embed, pos_embed, seg_embed, gamma, beta):
    table = _build_table(tok_embed, pos_embed[:_S], seg_embed, gamma, beta)
    xt = x.T.astype(jnp.int32)
    st = seg.T.astype(jnp.int32)
    out_t = _gather_rows(table, xt, st)
    return jnp.transpose(out_t, (1, 0, 2))
